# Initial kernel scaffold; baseline (speedup 1.0000x reference)
import jax
import jax.numpy as jnp
from jax.experimental import pallas as pl


def kernel(x, W0, b0, Wsg0, bsg0, Wg, We1, We3, We2, Ws1, Ws3, Ws2, W1, b1, Wsg1, bsg1, W2, b2, Wsg2, bsg2):
    raise NotImplementedError("write your pallas kernel here")


# speedup vs baseline: 1.3622x; 1.3622x over previous
import functools

import jax
import jax.numpy as jnp
from jax.experimental import pallas as pl
from jax.experimental.pallas import tpu as pltpu

N = 2048
D = 768
E = 8
TOPK = 4
HID = 768
SHID = 2 * 768

TILE = 256


def _silu(v):
    return v * jax.nn.sigmoid(v)


def _pre_body(x_ref, W0_ref, b0_ref, Wsg0_ref, bsg0_ref, Wg_ref,
              Ws1_ref, Ws3_ref, Ws2_ref,
              h_ref, comb_ref, S_ref):
    x = x_ref[...]
    t0 = jnp.dot(x, W0_ref[...], preferred_element_type=jnp.float32) + b0_ref[...]
    z = jnp.dot(t0, Wsg0_ref[...], preferred_element_type=jnp.float32) + bsg0_ref[...]
    a = z[:, :D]
    g = z[:, D:]
    h = a * _silu(g)
    h_ref[...] = h

    logits = jnp.dot(h, Wg_ref[...], preferred_element_type=jnp.float32)
    m = jnp.max(logits, axis=-1, keepdims=True)
    p = jnp.exp(logits - m)
    s = p / jnp.sum(p, axis=-1, keepdims=True)
    col = jax.lax.broadcasted_iota(jnp.int32, (TILE, E), 1)
    rank = jnp.zeros((TILE, E), jnp.int32)
    for j in range(E):
        sj = s[:, j:j + 1]
        gt = (sj > s).astype(jnp.int32)
        tie = ((sj == s) & (j < col)).astype(jnp.int32)
        rank = rank + gt + tie
    comb_ref[...] = jnp.where(rank < TOPK, s, 0.0)

    z1 = jnp.dot(h, Ws1_ref[...], preferred_element_type=jnp.float32)
    z3 = jnp.dot(h, Ws3_ref[...], preferred_element_type=jnp.float32)
    S_ref[...] = jnp.dot(_silu(z1) * z3, Ws2_ref[...],
                         preferred_element_type=jnp.float32)


def _moe_body(h_ref, comb_ref, S_ref, We1_ref, We3_ref, We2_ref, out_ref):
    e = pl.program_id(0)
    h = h_ref[...]
    h1 = jnp.dot(h, We1_ref[0], preferred_element_type=jnp.float32)
    h3 = jnp.dot(h, We3_ref[0], preferred_element_type=jnp.float32)
    eo = jnp.dot(_silu(h1) * h3, We2_ref[0], preferred_element_type=jnp.float32)
    lane = jax.lax.broadcasted_iota(jnp.int32, (1, E), 1)
    col = jnp.sum(comb_ref[...] * (lane == e).astype(jnp.float32),
                  axis=1, keepdims=True)
    contrib = col * eo

    @pl.when(e == 0)
    def _init():
        out_ref[...] = S_ref[...] + contrib

    @pl.when(e != 0)
    def _acc():
        out_ref[...] = out_ref[...] + contrib


def _post_body(y_ref, W1_ref, b1_ref, Wsg1_ref, bsg1_ref,
               W2_ref, b2_ref, Wsg2_ref, bsg2_ref, out_ref):
    y = y_ref[...]
    t1 = jnp.dot(y, W1_ref[...], preferred_element_type=jnp.float32) + b1_ref[...]
    z1 = jnp.dot(t1, Wsg1_ref[...], preferred_element_type=jnp.float32) + bsg1_ref[...]
    y1 = z1[:, :D] * _silu(z1[:, D:])
    t2 = jnp.dot(y1, W2_ref[...], preferred_element_type=jnp.float32) + b2_ref[...]
    z2 = jnp.dot(t2, Wsg2_ref[...], preferred_element_type=jnp.float32) + bsg2_ref[...]
    out_ref[...] = z2[:, :D] * _silu(z2[:, D:])


def kernel(x, W0, b0, Wsg0, bsg0, Wg, We1, We3, We2, Ws1, Ws3, Ws2,
           W1, b1, Wsg1, bsg1, W2, b2, Wsg2, bsg2):
    n_tiles = N // TILE

    full = lambda shape: pl.BlockSpec(shape, lambda t: (0,) * len(shape))
    row_tile = pl.BlockSpec((TILE, D), lambda t: (t, 0))

    h, comb, S = pl.pallas_call(
        _pre_body,
        grid=(n_tiles,),
        in_specs=[
            row_tile,
            full((D, D)), full((D,)), full((D, 2 * D)), full((2 * D,)),
            full((D, E)),
            full((D, SHID)), full((D, SHID)), full((SHID, D)),
        ],
        out_specs=[
            row_tile,
            pl.BlockSpec((TILE, E), lambda t: (t, 0)),
            row_tile,
        ],
        out_shape=[
            jax.ShapeDtypeStruct((N, D), jnp.float32),
            jax.ShapeDtypeStruct((N, E), jnp.float32),
            jax.ShapeDtypeStruct((N, D), jnp.float32),
        ],
        compiler_params=pltpu.CompilerParams(
            dimension_semantics=("arbitrary",),
        ),
    )(x, W0, b0, Wsg0, bsg0, Wg, Ws1, Ws3, Ws2)

    routed = pl.pallas_call(
        _moe_body,
        grid=(E,),
        in_specs=[
            pl.BlockSpec((N, D), lambda e: (0, 0)),
            pl.BlockSpec((N, E), lambda e: (0, 0)),
            pl.BlockSpec((N, D), lambda e: (0, 0)),
            pl.BlockSpec((1, D, HID), lambda e: (e, 0, 0)),
            pl.BlockSpec((1, D, HID), lambda e: (e, 0, 0)),
            pl.BlockSpec((1, HID, D), lambda e: (e, 0, 0)),
        ],
        out_specs=pl.BlockSpec((N, D), lambda e: (0, 0)),
        out_shape=jax.ShapeDtypeStruct((N, D), jnp.float32),
        compiler_params=pltpu.CompilerParams(
            dimension_semantics=("arbitrary",),
        ),
    )(h, comb, S, We1, We3, We2)

    out = pl.pallas_call(
        _post_body,
        grid=(n_tiles,),
        in_specs=[
            row_tile,
            full((D, D)), full((D,)), full((D, 2 * D)), full((2 * D,)),
            full((D, D)), full((D,)), full((D, 2 * D)), full((2 * D,)),
        ],
        out_specs=row_tile,
        out_shape=jax.ShapeDtypeStruct((N, D), jnp.float32),
        compiler_params=pltpu.CompilerParams(
            dimension_semantics=("arbitrary",),
        ),
    )(routed, W1, b1, Wsg1, bsg1, W2, b2, Wsg2, bsg2)

    return out
